# initial kernel scaffold (unmeasured)
import jax
import jax.numpy as jnp
from jax import lax
from jax.experimental import pallas as pl
from jax.experimental.pallas import tpu as pltpu

N_DEV = 4


def kernel(x, w_mat):
    m = x.shape[0]
    n = w_mat.shape[1]
    chunk = m // N_DEV

    xb = x.astype(jnp.bfloat16)
    wb = w_mat.astype(jnp.bfloat16)
    partial = jnp.dot(xb, wb, preferred_element_type=jnp.float32).astype(
        jnp.bfloat16
    )

    def body(
        p_ref,
        out_ref,
        send_buf,
        recv_buf,
        work_buf,
        out_stage,
        send_sem,
        recv_sem,
        in_sem,
        out_sem,
        credit_sem,
    ):
        d = lax.axis_index("i")
        left = lax.rem(d + N_DEV - 1, N_DEV)
        right = lax.rem(d + 1, N_DEV)

        barrier = pltpu.get_barrier_semaphore()
        for nbr in (left, right):
            pl.semaphore_signal(
                barrier,
                inc=1,
                device_id=(nbr,),
                device_id_type=pl.DeviceIdType.MESH,
            )
        pl.semaphore_wait(barrier, 2)

        def load_chunk(c, dst):
            cp = pltpu.make_async_copy(
                p_ref.at[pl.ds(c * chunk, chunk), :], dst, in_sem
            )
            cp.start()
            return cp

        def ring_send():
            rdma = pltpu.make_async_remote_copy(
                src_ref=send_buf,
                dst_ref=recv_buf,
                send_sem=send_sem,
                recv_sem=recv_sem,
                device_id=(right,),
                device_id_type=pl.DeviceIdType.MESH,
            )
            rdma.start()
            return rdma

        def credit_handshake():
            pl.semaphore_signal(
                credit_sem,
                inc=1,
                device_id=(left,),
                device_id_type=pl.DeviceIdType.MESH,
            )
            pl.semaphore_wait(credit_sem, 1)

        def store_chunk(c, buf):
            y = buf[...].astype(jnp.float32)
            out_stage[...] = y * (1.0 / (1.0 + jnp.exp(-y)))
            cp = pltpu.make_async_copy(
                out_stage, out_ref.at[pl.ds(c * chunk, chunk), :], out_sem
            )
            cp.start()
            cp.wait()

        load_chunk(d, send_buf).wait()

        for h in range(N_DEV - 1):
            rdma = ring_send()
            c_in = lax.rem(d + N_DEV - 1 - h, N_DEV)
            lcp = load_chunk(c_in, work_buf)
            rdma.wait()
            lcp.wait()
            send_buf[...] = (
                recv_buf[...].astype(jnp.float32)
                + work_buf[...].astype(jnp.float32)
            ).astype(jnp.bfloat16)
            credit_handshake()

        store_chunk(lax.rem(d + 1, N_DEV), send_buf)

        for h in range(N_DEV - 1):
            rdma = ring_send()
            rdma.wait()
            c = lax.rem(d + N_DEV - h, N_DEV)
            store_chunk(c, recv_buf)
            if h < N_DEV - 2:
                send_buf[...] = recv_buf[...]
            credit_handshake()

    return pl.pallas_call(
        body,
        out_shape=jax.ShapeDtypeStruct((m, n), jnp.float32),
        in_specs=[pl.BlockSpec(memory_space=pltpu.ANY)],
        out_specs=pl.BlockSpec(memory_space=pltpu.ANY),
        scratch_shapes=[
            pltpu.VMEM((chunk, n), jnp.bfloat16),
            pltpu.VMEM((chunk, n), jnp.bfloat16),
            pltpu.VMEM((chunk, n), jnp.bfloat16),
            pltpu.VMEM((chunk, n), jnp.float32),
            pltpu.SemaphoreType.DMA,
            pltpu.SemaphoreType.DMA,
            pltpu.SemaphoreType.DMA,
            pltpu.SemaphoreType.DMA,
            pltpu.SemaphoreType.REGULAR,
        ],
        compiler_params=pltpu.CompilerParams(collective_id=0),
    )(partial)


# baseline (device time: 1416863 ns/iter reference)
import jax
import jax.numpy as jnp
from jax import lax
from jax.experimental import pallas as pl
from jax.experimental.pallas import tpu as pltpu

N_DEV = 4
STAGE_ROWS = 256


def kernel(x, w_mat):
    m = x.shape[0]
    n = w_mat.shape[1]
    chunk = m // N_DEV
    hn = n // 2

    xb = x.astype(jnp.bfloat16)
    wb = w_mat.astype(jnp.bfloat16)
    partial = jnp.dot(xb, wb, preferred_element_type=jnp.float32).astype(
        jnp.bfloat16
    )

    def body(
        p_ref,
        out_ref,
        send_buf,
        recv_buf,
        work_buf,
        out_stage,
        send_sem,
        recv_sem,
        in_sem,
        out_sem,
        credit_sem,
    ):
        d = lax.axis_index("i")
        left = lax.rem(d + N_DEV - 1, N_DEV)
        right = lax.rem(d + 1, N_DEV)

        barrier = pltpu.get_barrier_semaphore()
        for nbr in (left, right):
            pl.semaphore_signal(
                barrier,
                inc=1,
                device_id=(nbr,),
                device_id_type=pl.DeviceIdType.MESH,
            )
        pl.semaphore_wait(barrier, 2)

        def ring_send():
            rdma = pltpu.make_async_remote_copy(
                src_ref=send_buf,
                dst_ref=recv_buf,
                send_sem=send_sem,
                recv_sem=recv_sem,
                device_id=(right,),
                device_id_type=pl.DeviceIdType.MESH,
            )
            rdma.start()
            return rdma

        def credit_handshake():
            pl.semaphore_signal(
                credit_sem,
                inc=1,
                device_id=(left,),
                device_id_type=pl.DeviceIdType.MESH,
            )
            pl.semaphore_wait(credit_sem, 1)

        for half in range(2):
            col = pl.ds(half * hn, hn)

            def load_chunk(c, dst):
                cp = pltpu.make_async_copy(
                    p_ref.at[pl.ds(c * chunk, chunk), col], dst, in_sem
                )
                cp.start()
                return cp

            def store_chunk(c, buf):
                for t in range(chunk // STAGE_ROWS):
                    rows = pl.ds(t * STAGE_ROWS, STAGE_ROWS)
                    y = buf[rows, :].astype(jnp.float32)
                    out_stage[...] = y * (1.0 / (1.0 + jnp.exp(-y)))
                    cp = pltpu.make_async_copy(
                        out_stage,
                        out_ref.at[pl.ds(c * chunk + t * STAGE_ROWS, STAGE_ROWS), col],
                        out_sem,
                    )
                    cp.start()
                    cp.wait()

            load_chunk(d, send_buf).wait()

            for h in range(N_DEV - 1):
                rdma = ring_send()
                c_in = lax.rem(d + N_DEV - 1 - h, N_DEV)
                lcp = load_chunk(c_in, work_buf)
                rdma.wait()
                lcp.wait()
                send_buf[...] = (
                    recv_buf[...].astype(jnp.float32)
                    + work_buf[...].astype(jnp.float32)
                ).astype(jnp.bfloat16)
                credit_handshake()

            store_chunk(lax.rem(d + 1, N_DEV), send_buf)

            for h in range(N_DEV - 1):
                rdma = ring_send()
                rdma.wait()
                c = lax.rem(d + N_DEV - h, N_DEV)
                store_chunk(c, recv_buf)
                if h < N_DEV - 2:
                    send_buf[...] = recv_buf[...]
                credit_handshake()

    return pl.pallas_call(
        body,
        out_shape=jax.ShapeDtypeStruct((m, n), jnp.float32),
        in_specs=[pl.BlockSpec(memory_space=pl.ANY)],
        out_specs=pl.BlockSpec(memory_space=pl.ANY),
        scratch_shapes=[
            pltpu.VMEM((chunk, hn), jnp.bfloat16),
            pltpu.VMEM((chunk, hn), jnp.bfloat16),
            pltpu.VMEM((chunk, hn), jnp.bfloat16),
            pltpu.VMEM((STAGE_ROWS, hn), jnp.float32),
            pltpu.SemaphoreType.DMA,
            pltpu.SemaphoreType.DMA,
            pltpu.SemaphoreType.DMA,
            pltpu.SemaphoreType.DMA,
            pltpu.SemaphoreType.REGULAR,
        ],
        compiler_params=pltpu.CompilerParams(collective_id=0),
    )(partial)


# device time: 899180 ns/iter; 1.5757x vs baseline; 1.5757x over previous
import jax
import jax.numpy as jnp
from jax import lax
from jax.experimental import pallas as pl
from jax.experimental.pallas import tpu as pltpu

N_DEV = 4
STAGE_ROWS = 256
N_PASS = 2


def kernel(x, w_mat):
    m = x.shape[0]
    n = w_mat.shape[1]
    chunk = m // N_DEV
    qn = n // 2 // N_PASS

    xb = x.astype(jnp.bfloat16)
    wb = w_mat.astype(jnp.bfloat16)
    partial = jnp.dot(xb, wb, preferred_element_type=jnp.float32).astype(
        jnp.bfloat16
    )

    def body(
        p_ref,
        out_ref,
        send_r,
        recv_r,
        work_r,
        send_l,
        recv_l,
        work_l,
        out_stage,
        send_sem_r,
        recv_sem_r,
        send_sem_l,
        recv_sem_l,
        in_sem_r,
        in_sem_l,
        out_sem,
        credit_r,
        credit_l,
    ):
        d = lax.axis_index("i")
        left = lax.rem(d + N_DEV - 1, N_DEV)
        right = lax.rem(d + 1, N_DEV)

        barrier = pltpu.get_barrier_semaphore()
        for nbr in (left, right):
            pl.semaphore_signal(
                barrier,
                inc=1,
                device_id=(nbr,),
                device_id_type=pl.DeviceIdType.MESH,
            )
        pl.semaphore_wait(barrier, 2)

        def start_rdma(src, dst, ssem, rsem, target):
            rdma = pltpu.make_async_remote_copy(
                src_ref=src,
                dst_ref=dst,
                send_sem=ssem,
                recv_sem=rsem,
                device_id=(target,),
                device_id_type=pl.DeviceIdType.MESH,
            )
            rdma.start()
            return rdma

        def credits():
            pl.semaphore_signal(
                credit_r,
                inc=1,
                device_id=(left,),
                device_id_type=pl.DeviceIdType.MESH,
            )
            pl.semaphore_signal(
                credit_l,
                inc=1,
                device_id=(right,),
                device_id_type=pl.DeviceIdType.MESH,
            )
            pl.semaphore_wait(credit_r, 1)
            pl.semaphore_wait(credit_l, 1)

        for p in range(N_PASS):
            col_r = pl.ds(p * qn, qn)
            col_l = pl.ds(n // 2 + p * qn, qn)

            def load(c, col, dst, sem):
                cp = pltpu.make_async_copy(
                    p_ref.at[pl.ds(c * chunk, chunk), col], dst, sem
                )
                cp.start()
                return cp

            def store_chunk(c, col, buf):
                for t in range(chunk // STAGE_ROWS):
                    rows = pl.ds(t * STAGE_ROWS, STAGE_ROWS)
                    y = buf[rows, :].astype(jnp.float32)
                    out_stage[...] = y * (1.0 / (1.0 + jnp.exp(-y)))
                    cp = pltpu.make_async_copy(
                        out_stage,
                        out_ref.at[
                            pl.ds(c * chunk + t * STAGE_ROWS, STAGE_ROWS), col
                        ],
                        out_sem,
                    )
                    cp.start()
                    cp.wait()

            cp_r = load(d, col_r, send_r, in_sem_r)
            cp_l = load(d, col_l, send_l, in_sem_l)
            cp_r.wait()
            cp_l.wait()

            for h in range(N_DEV - 1):
                rdma_r = start_rdma(send_r, recv_r, send_sem_r, recv_sem_r, right)
                rdma_l = start_rdma(send_l, recv_l, send_sem_l, recv_sem_l, left)
                c_in_r = lax.rem(d + N_DEV - 1 - h, N_DEV)
                c_in_l = lax.rem(d + 1 + h, N_DEV)
                lcp_r = load(c_in_r, col_r, work_r, in_sem_r)
                lcp_l = load(c_in_l, col_l, work_l, in_sem_l)
                rdma_r.wait()
                lcp_r.wait()
                send_r[...] = (
                    recv_r[...].astype(jnp.float32)
                    + work_r[...].astype(jnp.float32)
                ).astype(jnp.bfloat16)
                rdma_l.wait()
                lcp_l.wait()
                send_l[...] = (
                    recv_l[...].astype(jnp.float32)
                    + work_l[...].astype(jnp.float32)
                ).astype(jnp.bfloat16)
                credits()

            store_chunk(lax.rem(d + 1, N_DEV), col_r, send_r)
            store_chunk(lax.rem(d + N_DEV - 1, N_DEV), col_l, send_l)

            for h in range(N_DEV - 1):
                rdma_r = start_rdma(send_r, recv_r, send_sem_r, recv_sem_r, right)
                rdma_l = start_rdma(send_l, recv_l, send_sem_l, recv_sem_l, left)
                rdma_r.wait()
                c_r = lax.rem(d + N_DEV - h, N_DEV)
                store_chunk(c_r, col_r, recv_r)
                rdma_l.wait()
                c_l = lax.rem(d + h, N_DEV)
                store_chunk(c_l, col_l, recv_l)
                if h < N_DEV - 2:
                    send_r[...] = recv_r[...]
                    send_l[...] = recv_l[...]
                credits()

    comm = pltpu.VMEM((chunk, qn), jnp.bfloat16)
    return pl.pallas_call(
        body,
        out_shape=jax.ShapeDtypeStruct((m, n), jnp.float32),
        in_specs=[pl.BlockSpec(memory_space=pl.ANY)],
        out_specs=pl.BlockSpec(memory_space=pl.ANY),
        scratch_shapes=[
            comm,
            comm,
            comm,
            comm,
            comm,
            comm,
            pltpu.VMEM((STAGE_ROWS, qn), jnp.float32),
            pltpu.SemaphoreType.DMA,
            pltpu.SemaphoreType.DMA,
            pltpu.SemaphoreType.DMA,
            pltpu.SemaphoreType.DMA,
            pltpu.SemaphoreType.DMA,
            pltpu.SemaphoreType.DMA,
            pltpu.SemaphoreType.DMA,
            pltpu.SemaphoreType.REGULAR,
            pltpu.SemaphoreType.REGULAR,
        ],
        compiler_params=pltpu.CompilerParams(collective_id=0),
    )(partial)


# device time: 796540 ns/iter; 1.7788x vs baseline; 1.1289x over previous
import jax
import jax.numpy as jnp
from jax import lax
from jax.experimental import pallas as pl
from jax.experimental.pallas import tpu as pltpu

N_DEV = 4
STAGE_ROWS = 256
N_PASS = 2


def kernel(x, w_mat):
    m = x.shape[0]
    n = w_mat.shape[1]
    chunk = m // N_DEV
    qn = n // 2 // N_PASS

    xb = x.astype(jnp.bfloat16)
    wb = w_mat.astype(jnp.bfloat16)
    partial = jnp.dot(xb, wb, preferred_element_type=jnp.float32).astype(
        jnp.bfloat16
    )

    def body(
        p_ref,
        out_ref,
        s_r, r_r, w_r,
        s_l, r_l, w_l,
        stage_r, stage_l,
        send_sem_r, recv1_r, recv2_r, in_sem_r, out_sem_r,
        send_sem_l, recv1_l, recv2_l, in_sem_l, out_sem_l,
        cred_rbuf_r, cred_wbuf_r, cred_rbuf_l, cred_wbuf_l,
    ):
        d = lax.axis_index("i")
        left = lax.rem(d + N_DEV - 1, N_DEV)
        right = lax.rem(d + 1, N_DEV)

        R = dict(S=s_r, R=r_r, W=w_r, stage=stage_r, send_sem=send_sem_r,
                 recv1=recv1_r, recv2=recv2_r, in_sem=in_sem_r,
                 out_sem=out_sem_r, cred_r=cred_rbuf_r, cred_w=cred_wbuf_r,
                 tgt=right, up=left, sgn=-1)
        L = dict(S=s_l, R=r_l, W=w_l, stage=stage_l, send_sem=send_sem_l,
                 recv1=recv1_l, recv2=recv2_l, in_sem=in_sem_l,
                 out_sem=out_sem_l, cred_r=cred_rbuf_l, cred_w=cred_wbuf_l,
                 tgt=left, up=right, sgn=1)
        DIRS = (R, L)

        def ckn(D, k):
            return lax.rem(d + N_DEV + D["sgn"] * k, N_DEV)

        barrier = pltpu.get_barrier_semaphore()
        for nbr in (left, right):
            pl.semaphore_signal(
                barrier, inc=1, device_id=(nbr,),
                device_id_type=pl.DeviceIdType.MESH,
            )
        pl.semaphore_wait(barrier, 2)

        def signal(sem, dev):
            pl.semaphore_signal(
                sem, inc=1, device_id=(dev,),
                device_id_type=pl.DeviceIdType.MESH,
            )

        def start_rdma(D, src, dst, rsem):
            rdma = pltpu.make_async_remote_copy(
                src_ref=src, dst_ref=dst, send_sem=D["send_sem"],
                recv_sem=rsem, device_id=(D["tgt"],),
                device_id_type=pl.DeviceIdType.MESH,
            )
            rdma.start()
            return rdma

        for D in DIRS:
            signal(D["cred_r"], D["up"])

        for p in range(N_PASS):
            def col(D):
                base = (0 if D is R else n // 2) + p * qn
                return pl.ds(base, qn)

            def load(D, c, dst):
                cp = pltpu.make_async_copy(
                    p_ref.at[pl.ds(c * chunk, chunk), col(D)], dst,
                    D["in_sem"],
                )
                cp.start()
                return cp

            def store(D, c, buf):
                for t in range(chunk // STAGE_ROWS):
                    rows = pl.ds(t * STAGE_ROWS, STAGE_ROWS)
                    y = buf[rows, :].astype(jnp.float32)
                    D["stage"][...] = y * (1.0 / (1.0 + jnp.exp(-y)))
                    cp = pltpu.make_async_copy(
                        D["stage"],
                        out_ref.at[
                            pl.ds(c * chunk + t * STAGE_ROWS, STAGE_ROWS),
                            col(D),
                        ],
                        D["out_sem"],
                    )
                    cp.start()
                    cp.wait()

            loads = [load(D, d, D["S"]) for D in DIRS]
            for cp in loads:
                cp.wait()

            for h in range(N_DEV - 1):
                rd = {}
                for D in DIRS:
                    pl.semaphore_wait(D["cred_r"], 1)
                    rd[id(D)] = start_rdma(D, D["S"], D["R"], D["recv1"])
                lc = {id(D): load(D, ckn(D, h + 1), D["W"]) for D in DIRS}
                for D in DIRS:
                    rd[id(D)].wait_recv()
                    lc[id(D)].wait()
                    rd[id(D)].wait_send()
                    D["S"][...] = D["R"][...] + D["W"][...]
                    signal(D["cred_r"], D["up"])
                if h == N_DEV - 2:
                    for D in DIRS:
                        signal(D["cred_w"], D["up"])


            rd0 = {}
            for D in DIRS:
                pl.semaphore_wait(D["cred_r"], 1)
                rd0[id(D)] = start_rdma(D, D["S"], D["R"], D["recv1"])
            for D in DIRS:
                store(D, ckn(D, N_DEV - 1), D["S"])
            for D in DIRS:
                rd0[id(D)].wait_recv()
                rd0[id(D)].wait_send()
            rd1 = {}
            for D in DIRS:
                pl.semaphore_wait(D["cred_w"], 1)
                rd1[id(D)] = start_rdma(D, D["R"], D["W"], D["recv2"])
            for D in DIRS:
                store(D, ckn(D, 0), D["R"])
            for D in DIRS:
                rd1[id(D)].wait_send()
                signal(D["cred_r"], D["up"])
            for D in DIRS:
                rd1[id(D)].wait_recv()
            rd2 = {}
            for D in DIRS:
                pl.semaphore_wait(D["cred_r"], 1)
                rd2[id(D)] = start_rdma(D, D["W"], D["R"], D["recv1"])
            for D in DIRS:
                store(D, ckn(D, 1), D["W"])
            for D in DIRS:
                rd2[id(D)].wait_send()
                rd2[id(D)].wait_recv()
            for D in DIRS:
                store(D, ckn(D, 2), D["R"])
                if p < N_PASS - 1:
                    signal(D["cred_r"], D["up"])

    comm = pltpu.VMEM((chunk, qn), jnp.bfloat16)
    return pl.pallas_call(
        body,
        out_shape=jax.ShapeDtypeStruct((m, n), jnp.float32),
        in_specs=[pl.BlockSpec(memory_space=pl.ANY)],
        out_specs=pl.BlockSpec(memory_space=pl.ANY),
        scratch_shapes=[
            comm, comm, comm,
            comm, comm, comm,
            pltpu.VMEM((STAGE_ROWS, qn), jnp.float32),
            pltpu.VMEM((STAGE_ROWS, qn), jnp.float32),
            pltpu.SemaphoreType.DMA,
            pltpu.SemaphoreType.DMA,
            pltpu.SemaphoreType.DMA,
            pltpu.SemaphoreType.DMA,
            pltpu.SemaphoreType.DMA,
            pltpu.SemaphoreType.DMA,
            pltpu.SemaphoreType.DMA,
            pltpu.SemaphoreType.DMA,
            pltpu.SemaphoreType.DMA,
            pltpu.SemaphoreType.DMA,
            pltpu.SemaphoreType.REGULAR,
            pltpu.SemaphoreType.REGULAR,
            pltpu.SemaphoreType.REGULAR,
            pltpu.SemaphoreType.REGULAR,
        ],
        compiler_params=pltpu.CompilerParams(collective_id=0),
    )(partial)


# device time: 742863 ns/iter; 1.9073x vs baseline; 1.0723x over previous
import jax
import jax.numpy as jnp
from jax import lax
from jax.experimental import pallas as pl
from jax.experimental.pallas import tpu as pltpu

N_DEV = 4
STAGE_ROWS = 256
N_PASS = 2


def kernel(x, w_mat):
    m = x.shape[0]
    k = x.shape[1]
    n = w_mat.shape[1]
    chunk = m // N_DEV
    qn = n // 2 // N_PASS

    xb = x.astype(jnp.bfloat16)
    wb = w_mat.astype(jnp.bfloat16)

    def body(
        x_ref,
        w_ref,
        out_ref,
        s_r, r_r, w_r,
        s_l, r_l, w_l,
        xbuf, wbuf, stage,
        send_sem_r, recv1_r, recv2_r,
        send_sem_l, recv1_l, recv2_l,
        in_x_sem, in_w_sem, out_sem,
        cred_rbuf_r, cred_wbuf_r, cred_rbuf_l, cred_wbuf_l,
    ):
        d = lax.axis_index("i")
        left = lax.rem(d + N_DEV - 1, N_DEV)
        right = lax.rem(d + 1, N_DEV)

        R = dict(S=s_r, R=r_r, W=w_r, send_sem=send_sem_r,
                 recv1=recv1_r, recv2=recv2_r,
                 cred_r=cred_rbuf_r, cred_w=cred_wbuf_r,
                 tgt=right, up=left, sgn=-1)
        L = dict(S=s_l, R=r_l, W=w_l, send_sem=send_sem_l,
                 recv1=recv1_l, recv2=recv2_l,
                 cred_r=cred_rbuf_l, cred_w=cred_wbuf_l,
                 tgt=left, up=right, sgn=1)
        DIRS = (R, L)

        def ckn(D, k_):
            return lax.rem(d + N_DEV + D["sgn"] * k_, N_DEV)

        barrier = pltpu.get_barrier_semaphore()
        for nbr in (left, right):
            pl.semaphore_signal(
                barrier, inc=1, device_id=(nbr,),
                device_id_type=pl.DeviceIdType.MESH,
            )
        pl.semaphore_wait(barrier, 2)

        def signal(sem, dev):
            pl.semaphore_signal(
                sem, inc=1, device_id=(dev,),
                device_id_type=pl.DeviceIdType.MESH,
            )

        def start_rdma(D, src, dst, rsem):
            rdma = pltpu.make_async_remote_copy(
                src_ref=src, dst_ref=dst, send_sem=D["send_sem"],
                recv_sem=rsem, device_id=(D["tgt"],),
                device_id_type=pl.DeviceIdType.MESH,
            )
            rdma.start()
            return rdma

        for D in DIRS:
            signal(D["cred_r"], D["up"])

        for p in range(N_PASS):
            def col(D):
                base = (0 if D is R else n // 2) + p * qn
                return pl.ds(base, qn)

            def compute(D, c, dst):
                cx = pltpu.make_async_copy(
                    x_ref.at[pl.ds(c * chunk, chunk), :], xbuf, in_x_sem
                )
                cx.start()
                base = (0 if D is R else n // 2) + p * qn
                for sub in range(2):
                    cw = pltpu.make_async_copy(
                        w_ref.at[:, pl.ds(base + sub * (qn // 2), qn // 2)],
                        wbuf,
                        in_w_sem,
                    )
                    cw.start()
                    if sub == 0:
                        cx.wait()
                    cw.wait()
                    dst[:, pl.ds(sub * (qn // 2), qn // 2)] = jnp.dot(
                        xbuf[...], wbuf[...],
                        preferred_element_type=jnp.float32,
                    ).astype(jnp.bfloat16)

            def store(D, c, buf):
                for t in range(chunk // STAGE_ROWS):
                    rows = pl.ds(t * STAGE_ROWS, STAGE_ROWS)
                    y = buf[rows, :].astype(jnp.float32)
                    stage[...] = y * (1.0 / (1.0 + jnp.exp(-y)))
                    cp = pltpu.make_async_copy(
                        stage,
                        out_ref.at[
                            pl.ds(c * chunk + t * STAGE_ROWS, STAGE_ROWS),
                            col(D),
                        ],
                        out_sem,
                    )
                    cp.start()
                    cp.wait()

            for D in DIRS:
                compute(D, d, D["S"])

            for h in range(N_DEV - 1):
                rd = {}
                for D in DIRS:
                    pl.semaphore_wait(D["cred_r"], 1)
                    rd[id(D)] = start_rdma(D, D["S"], D["R"], D["recv1"])
                for D in DIRS:
                    compute(D, ckn(D, h + 1), D["W"])
                for D in DIRS:
                    rd[id(D)].wait_recv()
                    rd[id(D)].wait_send()
                    D["S"][...] = D["R"][...] + D["W"][...]
                    signal(D["cred_r"], D["up"])
                if h == N_DEV - 2:
                    for D in DIRS:
                        signal(D["cred_w"], D["up"])


            rd0 = {}
            for D in DIRS:
                pl.semaphore_wait(D["cred_r"], 1)
                rd0[id(D)] = start_rdma(D, D["S"], D["R"], D["recv1"])
            for D in DIRS:
                store(D, ckn(D, N_DEV - 1), D["S"])
            for D in DIRS:
                rd0[id(D)].wait_recv()
                rd0[id(D)].wait_send()
            rd1 = {}
            for D in DIRS:
                pl.semaphore_wait(D["cred_w"], 1)
                rd1[id(D)] = start_rdma(D, D["R"], D["W"], D["recv2"])
            for D in DIRS:
                store(D, ckn(D, 0), D["R"])
            for D in DIRS:
                rd1[id(D)].wait_send()
                signal(D["cred_r"], D["up"])
            for D in DIRS:
                rd1[id(D)].wait_recv()
            rd2 = {}
            for D in DIRS:
                pl.semaphore_wait(D["cred_r"], 1)
                rd2[id(D)] = start_rdma(D, D["W"], D["R"], D["recv1"])
            for D in DIRS:
                store(D, ckn(D, 1), D["W"])
            for D in DIRS:
                rd2[id(D)].wait_send()
                rd2[id(D)].wait_recv()
            for D in DIRS:
                store(D, ckn(D, 2), D["R"])
                if p < N_PASS - 1:
                    signal(D["cred_r"], D["up"])

    comm = pltpu.VMEM((chunk, qn), jnp.bfloat16)
    return pl.pallas_call(
        body,
        out_shape=jax.ShapeDtypeStruct((m, n), jnp.float32),
        in_specs=[
            pl.BlockSpec(memory_space=pl.ANY),
            pl.BlockSpec(memory_space=pl.ANY),
        ],
        out_specs=pl.BlockSpec(memory_space=pl.ANY),
        scratch_shapes=[
            comm, comm, comm,
            comm, comm, comm,
            pltpu.VMEM((chunk, k), jnp.bfloat16),
            pltpu.VMEM((k, qn // 2), jnp.bfloat16),
            pltpu.VMEM((STAGE_ROWS, qn), jnp.float32),
            pltpu.SemaphoreType.DMA,
            pltpu.SemaphoreType.DMA,
            pltpu.SemaphoreType.DMA,
            pltpu.SemaphoreType.DMA,
            pltpu.SemaphoreType.DMA,
            pltpu.SemaphoreType.DMA,
            pltpu.SemaphoreType.DMA,
            pltpu.SemaphoreType.DMA,
            pltpu.SemaphoreType.DMA,
            pltpu.SemaphoreType.REGULAR,
            pltpu.SemaphoreType.REGULAR,
            pltpu.SemaphoreType.REGULAR,
            pltpu.SemaphoreType.REGULAR,
        ],
        compiler_params=pltpu.CompilerParams(collective_id=0),
    )(xb, wb)


# device time: 739832 ns/iter; 1.9151x vs baseline; 1.0041x over previous
import jax
import jax.numpy as jnp
from jax import lax
from jax.experimental import pallas as pl
from jax.experimental.pallas import tpu as pltpu

N_DEV = 4
STAGE_ROWS = 256
N_PASS = 2


def kernel(x, w_mat):
    m = x.shape[0]
    k = x.shape[1]
    n = w_mat.shape[1]
    chunk = m // N_DEV
    qn = n // 2 // N_PASS

    xb = x.astype(jnp.bfloat16)
    wb = w_mat.astype(jnp.bfloat16)

    def body(
        x_ref,
        w_ref,
        out_ref,
        s_r, r_r, w_r,
        s_l, r_l, w_l,
        xbuf, wbuf, stage,
        send_sem_r, send2_sem_r, recv1_r, recv2_r,
        send_sem_l, send2_sem_l, recv1_l, recv2_l,
        in_x_sem, in_w_sem, out_sem,
        cred_rbuf_r, cred_bbuf_r, cred_wbuf_r,
        cred_rbuf_l, cred_bbuf_l, cred_wbuf_l,
    ):
        d = lax.axis_index("i")
        left = lax.rem(d + N_DEV - 1, N_DEV)
        right = lax.rem(d + 1, N_DEV)

        R = dict(S=s_r, R=r_r, W=w_r, send_sem=send_sem_r,
                 send2=send2_sem_r, recv1=recv1_r, recv2=recv2_r,
                 cred_r=cred_rbuf_r, cred_b=cred_bbuf_r, cred_w=cred_wbuf_r,
                 tgt=right, up=left, sgn=-1)
        L = dict(S=s_l, R=r_l, W=w_l, send_sem=send_sem_l,
                 send2=send2_sem_l, recv1=recv1_l, recv2=recv2_l,
                 cred_r=cred_rbuf_l, cred_b=cred_bbuf_l, cred_w=cred_wbuf_l,
                 tgt=left, up=right, sgn=1)
        DIRS = (R, L)

        def ckn(D, k_):
            return lax.rem(d + N_DEV + D["sgn"] * k_, N_DEV)

        barrier = pltpu.get_barrier_semaphore()
        for nbr in (left, right):
            pl.semaphore_signal(
                barrier, inc=1, device_id=(nbr,),
                device_id_type=pl.DeviceIdType.MESH,
            )
        pl.semaphore_wait(barrier, 2)

        def signal(sem, dev):
            pl.semaphore_signal(
                sem, inc=1, device_id=(dev,),
                device_id_type=pl.DeviceIdType.MESH,
            )

        def start_rdma(D, src, dst, rsem):
            rdma = pltpu.make_async_remote_copy(
                src_ref=src, dst_ref=dst, send_sem=D["send_sem"],
                recv_sem=rsem, device_id=(D["tgt"],),
                device_id_type=pl.DeviceIdType.MESH,
            )
            rdma.start()
            return rdma

        for D in DIRS:
            signal(D["cred_r"], D["up"])
            signal(D["cred_b"], D["up"])

        hw = qn // 2

        def start_sub(D, i):
            sl = pl.ds(i * hw, hw)
            rdma = pltpu.make_async_remote_copy(
                src_ref=D["S"].at[:, sl],
                dst_ref=D["R"].at[:, sl],
                send_sem=D["send_sem"] if i == 0 else D["send2"],
                recv_sem=D["recv1"] if i == 0 else D["recv2"],
                device_id=(D["tgt"],),
                device_id_type=pl.DeviceIdType.MESH,
            )
            rdma.start()
            return rdma

        for p in range(N_PASS):
            def col(D):
                base = (0 if D is R else n // 2) + p * qn
                return pl.ds(base, qn)

            def compute(D, c, dst):
                cx = pltpu.make_async_copy(
                    x_ref.at[pl.ds(c * chunk, chunk), :], xbuf, in_x_sem
                )
                cx.start()
                base = (0 if D is R else n // 2) + p * qn
                for sub in range(2):
                    cw = pltpu.make_async_copy(
                        w_ref.at[:, pl.ds(base + sub * (qn // 2), qn // 2)],
                        wbuf,
                        in_w_sem,
                    )
                    cw.start()
                    if sub == 0:
                        cx.wait()
                    cw.wait()
                    dst[:, pl.ds(sub * (qn // 2), qn // 2)] = jnp.dot(
                        xbuf[...], wbuf[...],
                        preferred_element_type=jnp.float32,
                    ).astype(jnp.bfloat16)

            def store(D, c, buf):
                for t in range(chunk // STAGE_ROWS):
                    rows = pl.ds(t * STAGE_ROWS, STAGE_ROWS)
                    y = buf[rows, :].astype(jnp.float32)
                    stage[...] = y * (1.0 / (1.0 + jnp.exp(-y)))
                    cp = pltpu.make_async_copy(
                        stage,
                        out_ref.at[
                            pl.ds(c * chunk + t * STAGE_ROWS, STAGE_ROWS),
                            col(D),
                        ],
                        out_sem,
                    )
                    cp.start()
                    cp.wait()

            for D in DIRS:
                compute(D, d, D["S"])

            for h in range(N_DEV - 1):
                rdA, rdB = {}, {}
                for D in DIRS:
                    pl.semaphore_wait(D["cred_r"], 1)
                    rdA[id(D)] = start_sub(D, 0)
                    pl.semaphore_wait(D["cred_b"], 1)
                    rdB[id(D)] = start_sub(D, 1)
                for D in DIRS:
                    compute(D, ckn(D, h + 1), D["W"])
                slA = pl.ds(0, hw)
                slB = pl.ds(hw, hw)
                for D in DIRS:
                    rdA[id(D)].wait_recv()
                    rdA[id(D)].wait_send()
                    D["S"][:, slA] = D["R"][:, slA] + D["W"][:, slA]
                    signal(D["cred_r"], D["up"])
                for D in DIRS:
                    rdB[id(D)].wait_recv()
                    rdB[id(D)].wait_send()
                    D["S"][:, slB] = D["R"][:, slB] + D["W"][:, slB]
                    signal(D["cred_b"], D["up"])
                if h == N_DEV - 2:
                    for D in DIRS:
                        signal(D["cred_w"], D["up"])


            rd0 = {}
            for D in DIRS:
                pl.semaphore_wait(D["cred_r"], 1)
                pl.semaphore_wait(D["cred_b"], 1)
                rd0[id(D)] = start_rdma(D, D["S"], D["R"], D["recv1"])
            for D in DIRS:
                store(D, ckn(D, N_DEV - 1), D["S"])
            for D in DIRS:
                rd0[id(D)].wait_recv()
                rd0[id(D)].wait_send()
            rd1 = {}
            for D in DIRS:
                pl.semaphore_wait(D["cred_w"], 1)
                rd1[id(D)] = start_rdma(D, D["R"], D["W"], D["recv2"])
            for D in DIRS:
                store(D, ckn(D, 0), D["R"])
            for D in DIRS:
                rd1[id(D)].wait_send()
                signal(D["cred_r"], D["up"])
                signal(D["cred_b"], D["up"])
            for D in DIRS:
                rd1[id(D)].wait_recv()
            rd2 = {}
            for D in DIRS:
                pl.semaphore_wait(D["cred_r"], 1)
                pl.semaphore_wait(D["cred_b"], 1)
                rd2[id(D)] = start_rdma(D, D["W"], D["R"], D["recv1"])
            for D in DIRS:
                store(D, ckn(D, 1), D["W"])
            for D in DIRS:
                rd2[id(D)].wait_send()
                rd2[id(D)].wait_recv()
            for D in DIRS:
                store(D, ckn(D, 2), D["R"])
                if p < N_PASS - 1:
                    signal(D["cred_r"], D["up"])
                    signal(D["cred_b"], D["up"])

    comm = pltpu.VMEM((chunk, qn), jnp.bfloat16)
    return pl.pallas_call(
        body,
        out_shape=jax.ShapeDtypeStruct((m, n), jnp.float32),
        in_specs=[
            pl.BlockSpec(memory_space=pl.ANY),
            pl.BlockSpec(memory_space=pl.ANY),
        ],
        out_specs=pl.BlockSpec(memory_space=pl.ANY),
        scratch_shapes=[
            comm, comm, comm,
            comm, comm, comm,
            pltpu.VMEM((chunk, k), jnp.bfloat16),
            pltpu.VMEM((k, qn // 2), jnp.bfloat16),
            pltpu.VMEM((STAGE_ROWS, qn), jnp.float32),
            pltpu.SemaphoreType.DMA,
            pltpu.SemaphoreType.DMA,
            pltpu.SemaphoreType.DMA,
            pltpu.SemaphoreType.DMA,
            pltpu.SemaphoreType.DMA,
            pltpu.SemaphoreType.DMA,
            pltpu.SemaphoreType.DMA,
            pltpu.SemaphoreType.DMA,
            pltpu.SemaphoreType.DMA,
            pltpu.SemaphoreType.DMA,
            pltpu.SemaphoreType.DMA,
            pltpu.SemaphoreType.REGULAR,
            pltpu.SemaphoreType.REGULAR,
            pltpu.SemaphoreType.REGULAR,
            pltpu.SemaphoreType.REGULAR,
            pltpu.SemaphoreType.REGULAR,
            pltpu.SemaphoreType.REGULAR,
        ],
        compiler_params=pltpu.CompilerParams(collective_id=0),
    )(xb, wb)


# device time: 718221 ns/iter; 1.9727x vs baseline; 1.0301x over previous
import jax
import jax.numpy as jnp
from jax import lax
from jax.experimental import pallas as pl
from jax.experimental.pallas import tpu as pltpu

N_DEV = 4
STAGE_ROWS = 256
N_PASS = 2


def kernel(x, w_mat):
    m = x.shape[0]
    k = x.shape[1]
    n = w_mat.shape[1]
    chunk = m // N_DEV
    qn = n // 2 // N_PASS

    xb = x.astype(jnp.bfloat16)
    wb = w_mat.astype(jnp.bfloat16)

    def body(
        x_ref,
        w_ref,
        out_ref,
        s_r, r_r, w_r,
        s_l, r_l, w_l,
        xbuf, wbuf, stage,
        send_sem_r, send2_sem_r, recv1_r, recv2_r,
        send_sem_l, send2_sem_l, recv1_l, recv2_l,
        in_x_sem, in_w_sem, out_sem,
        cred_rbuf_r, cred_bbuf_r, cred_wbuf_r,
        cred_rbuf_l, cred_bbuf_l, cred_wbuf_l,
    ):
        d = lax.axis_index("i")
        left = lax.rem(d + N_DEV - 1, N_DEV)
        right = lax.rem(d + 1, N_DEV)

        R = dict(S=s_r, R=r_r, W=w_r, send_sem=send_sem_r,
                 send2=send2_sem_r, recv1=recv1_r, recv2=recv2_r,
                 cred_r=cred_rbuf_r, cred_b=cred_bbuf_r, cred_w=cred_wbuf_r,
                 tgt=right, up=left, sgn=-1)
        L = dict(S=s_l, R=r_l, W=w_l, send_sem=send_sem_l,
                 send2=send2_sem_l, recv1=recv1_l, recv2=recv2_l,
                 cred_r=cred_rbuf_l, cred_b=cred_bbuf_l, cred_w=cred_wbuf_l,
                 tgt=left, up=right, sgn=1)
        DIRS = (R, L)

        def ckn(D, k_):
            return lax.rem(d + N_DEV + D["sgn"] * k_, N_DEV)

        barrier = pltpu.get_barrier_semaphore()
        for nbr in (left, right):
            pl.semaphore_signal(
                barrier, inc=1, device_id=(nbr,),
                device_id_type=pl.DeviceIdType.MESH,
            )
        pl.semaphore_wait(barrier, 2)

        def signal(sem, dev):
            pl.semaphore_signal(
                sem, inc=1, device_id=(dev,),
                device_id_type=pl.DeviceIdType.MESH,
            )

        def start_rdma(D, src, dst, rsem):
            rdma = pltpu.make_async_remote_copy(
                src_ref=src, dst_ref=dst, send_sem=D["send_sem"],
                recv_sem=rsem, device_id=(D["tgt"],),
                device_id_type=pl.DeviceIdType.MESH,
            )
            rdma.start()
            return rdma

        for D in DIRS:
            signal(D["cred_r"], D["up"])
            signal(D["cred_b"], D["up"])

        hw = qn // 2

        def start_sub(D, i, src=None):
            sl = pl.ds(i * hw, hw)
            rdma = pltpu.make_async_remote_copy(
                src_ref=(D["S"] if src is None else src).at[:, sl],
                dst_ref=D["R"].at[:, sl],
                send_sem=D["send_sem"] if i == 0 else D["send2"],
                recv_sem=D["recv1"] if i == 0 else D["recv2"],
                device_id=(D["tgt"],),
                device_id_type=pl.DeviceIdType.MESH,
            )
            rdma.start()
            return rdma

        for p in range(N_PASS):
            def col(D):
                base = (0 if D is R else n // 2) + p * qn
                return pl.ds(base, qn)

            def compute(D, c, dst, pp=p):
                cx = pltpu.make_async_copy(
                    x_ref.at[pl.ds(c * chunk, chunk), :], xbuf, in_x_sem
                )
                cx.start()
                base = (0 if D is R else n // 2) + pp * qn
                for sub in range(2):
                    cw = pltpu.make_async_copy(
                        w_ref.at[:, pl.ds(base + sub * (qn // 2), qn // 2)],
                        wbuf,
                        in_w_sem,
                    )
                    cw.start()
                    if sub == 0:
                        cx.wait()
                    cw.wait()
                    dst[:, pl.ds(sub * (qn // 2), qn // 2)] = jnp.dot(
                        xbuf[...], wbuf[...],
                        preferred_element_type=jnp.float32,
                    ).astype(jnp.bfloat16)

            def store(D, c, buf):
                for t in range(chunk // STAGE_ROWS):
                    rows = pl.ds(t * STAGE_ROWS, STAGE_ROWS)
                    y = buf[rows, :].astype(jnp.float32)
                    stage[...] = y * (1.0 / (1.0 + jnp.exp(-y)))
                    cp = pltpu.make_async_copy(
                        stage,
                        out_ref.at[
                            pl.ds(c * chunk + t * STAGE_ROWS, STAGE_ROWS),
                            col(D),
                        ],
                        out_sem,
                    )
                    cp.start()
                    cp.wait()

            def store_half(D, c, buf, i):
                base = (0 if D is R else n // 2) + p * qn + i * hw
                sl = pl.ds(i * hw, hw)
                for t in range(chunk // STAGE_ROWS):
                    rows = pl.ds(t * STAGE_ROWS, STAGE_ROWS)
                    y = buf[rows, sl].astype(jnp.float32)
                    stage[:, pl.ds(0, hw)] = y * (1.0 / (1.0 + jnp.exp(-y)))
                    cp = pltpu.make_async_copy(
                        stage.at[:, pl.ds(0, hw)],
                        out_ref.at[
                            pl.ds(c * chunk + t * STAGE_ROWS, STAGE_ROWS),
                            pl.ds(base, hw),
                        ],
                        out_sem,
                    )
                    cp.start()
                    cp.wait()

            if p == 0:
                for D in DIRS:
                    compute(D, d, D["S"])

            for h in range(N_DEV - 1):
                rdA, rdB = {}, {}
                for D in DIRS:
                    pl.semaphore_wait(D["cred_r"], 1)
                    rdA[id(D)] = start_sub(D, 0)
                    pl.semaphore_wait(D["cred_b"], 1)
                    rdB[id(D)] = start_sub(D, 1)
                for D in DIRS:
                    compute(D, ckn(D, h + 1), D["W"])
                slA = pl.ds(0, hw)
                slB = pl.ds(hw, hw)
                for D in DIRS:
                    rdA[id(D)].wait_recv()
                    rdA[id(D)].wait_send()
                    D["S"][:, slA] = D["R"][:, slA] + D["W"][:, slA]
                    signal(D["cred_r"], D["up"])
                for D in DIRS:
                    rdB[id(D)].wait_recv()
                    rdB[id(D)].wait_send()
                    D["S"][:, slB] = D["R"][:, slB] + D["W"][:, slB]
                    signal(D["cred_b"], D["up"])
                if h == N_DEV - 2:
                    for D in DIRS:
                        signal(D["cred_w"], D["up"])


            rd0 = {}
            for D in DIRS:
                pl.semaphore_wait(D["cred_r"], 1)
                pl.semaphore_wait(D["cred_b"], 1)
                rd0[id(D)] = start_rdma(D, D["S"], D["R"], D["recv1"])
            for D in DIRS:
                store(D, ckn(D, N_DEV - 1), D["S"])
            for D in DIRS:
                rd0[id(D)].wait_recv()
                rd0[id(D)].wait_send()
            rd1 = {}
            for D in DIRS:
                pl.semaphore_wait(D["cred_w"], 1)
                rd1[id(D)] = start_rdma(D, D["R"], D["W"], D["recv2"])
            for D in DIRS:
                store(D, ckn(D, 0), D["R"])
            for D in DIRS:
                rd1[id(D)].wait_send()
                signal(D["cred_r"], D["up"])
                signal(D["cred_b"], D["up"])
            for D in DIRS:
                rd1[id(D)].wait_recv()
            rd2A, rd2B = {}, {}
            for D in DIRS:
                pl.semaphore_wait(D["cred_r"], 1)
                rd2A[id(D)] = start_sub(D, 0, src=D["W"])
                pl.semaphore_wait(D["cred_b"], 1)
                rd2B[id(D)] = start_sub(D, 1, src=D["W"])
            for D in DIRS:
                store(D, ckn(D, 1), D["W"])
            for D in DIRS:
                rd2A[id(D)].wait_recv()
            for D in DIRS:
                store_half(D, ckn(D, 2), D["R"], 0)
            if p < N_PASS - 1:
                for D in DIRS:
                    compute(D, d, D["S"], p + 1)
            for D in DIRS:
                rd2B[id(D)].wait_recv()
                rd2A[id(D)].wait_send()
                rd2B[id(D)].wait_send()
            for D in DIRS:
                store_half(D, ckn(D, 2), D["R"], 1)
                if p < N_PASS - 1:
                    signal(D["cred_r"], D["up"])
                    signal(D["cred_b"], D["up"])

    comm = pltpu.VMEM((chunk, qn), jnp.bfloat16)
    return pl.pallas_call(
        body,
        out_shape=jax.ShapeDtypeStruct((m, n), jnp.float32),
        in_specs=[
            pl.BlockSpec(memory_space=pl.ANY),
            pl.BlockSpec(memory_space=pl.ANY),
        ],
        out_specs=pl.BlockSpec(memory_space=pl.ANY),
        scratch_shapes=[
            comm, comm, comm,
            comm, comm, comm,
            pltpu.VMEM((chunk, k), jnp.bfloat16),
            pltpu.VMEM((k, qn // 2), jnp.bfloat16),
            pltpu.VMEM((STAGE_ROWS, qn), jnp.float32),
            pltpu.SemaphoreType.DMA,
            pltpu.SemaphoreType.DMA,
            pltpu.SemaphoreType.DMA,
            pltpu.SemaphoreType.DMA,
            pltpu.SemaphoreType.DMA,
            pltpu.SemaphoreType.DMA,
            pltpu.SemaphoreType.DMA,
            pltpu.SemaphoreType.DMA,
            pltpu.SemaphoreType.DMA,
            pltpu.SemaphoreType.DMA,
            pltpu.SemaphoreType.DMA,
            pltpu.SemaphoreType.REGULAR,
            pltpu.SemaphoreType.REGULAR,
            pltpu.SemaphoreType.REGULAR,
            pltpu.SemaphoreType.REGULAR,
            pltpu.SemaphoreType.REGULAR,
            pltpu.SemaphoreType.REGULAR,
        ],
        compiler_params=pltpu.CompilerParams(collective_id=0),
    )(xb, wb)
